# Initial kernel scaffold; baseline (speedup 1.0000x reference)
import jax
import jax.numpy as jnp
from jax.experimental import pallas as pl


def kernel(x, edge_index, eps, W1, b1, W2, b2):
    raise NotImplementedError("write your pallas kernel here")


# speedup vs baseline: 4.4770x; 4.4770x over previous
import functools

import jax
import jax.numpy as jnp
from jax import lax
from jax.experimental import pallas as pl
from jax.experimental.pallas import tpu as pltpu
from jax.experimental.pallas import tpu_sc as plsc

N = 10000
E = 320000
D = 128

NC = 2
NS = 16
NW = NC * NS

CHUNK = 128
CHUNKS_PER_TILE = 79
EDGES_PER_TILE = CHUNK * CHUNKS_PER_TILE
E_PAD = EDGES_PER_TILE * NW
N_PAD = 10112
ZROWS = N_PAD // NS


def _sc_aggregate(src_hbm, dst_hbm, x_hbm, z_hbm, out_hbm,
                  acc, sidx, didx, rows, sem):
    cid = lax.axis_index("c")
    sid = lax.axis_index("s")
    wid = cid * NS + sid

    pltpu.sync_copy(z_hbm, acc.at[pl.ds(sid * ZROWS, ZROWS)])
    plsc.subcore_barrier()

    base = wid * EDGES_PER_TILE

    def step(i, carry):
        off = pl.multiple_of(base + i * CHUNK, CHUNK)
        pltpu.sync_copy(src_hbm.at[pl.ds(off, CHUNK)], sidx)
        pltpu.sync_copy(dst_hbm.at[pl.ds(off, CHUNK)], didx)
        pltpu.async_copy(x_hbm.at[sidx], rows, sem).wait()
        pltpu.sync_copy(rows, acc.at[didx], add=True)
        return carry

    lax.fori_loop(0, CHUNKS_PER_TILE, step, 0)
    plsc.subcore_barrier()

    row0 = sid * ZROWS
    out_off = pl.multiple_of(cid * N_PAD + row0, ZROWS)
    pltpu.sync_copy(acc.at[pl.ds(row0, ZROWS)],
                    out_hbm.at[pl.ds(out_off, ZROWS)])


_sc_call = functools.partial(
    pl.kernel,
    out_type=jax.ShapeDtypeStruct((NC * N_PAD, D), jnp.float32),
    mesh=plsc.VectorSubcoreMesh(core_axis_name="c", subcore_axis_name="s",
                                num_cores=NC, num_subcores=NS),
    scratch_types=[
        pltpu.VMEM_SHARED((N_PAD, D), jnp.float32),
        pltpu.VMEM((CHUNK,), jnp.int32),
        pltpu.VMEM((CHUNK,), jnp.int32),
        pltpu.VMEM((CHUNK, D), jnp.float32),
        pltpu.SemaphoreType.DMA,
    ],
)(_sc_aggregate)


def _tc_mlp(eps_ref, x_ref, p_ref, w1_ref, b1_ref, w2_ref, b2_ref, o_ref):
    h = x_ref[...] * (1.0 + eps_ref[0]) + p_ref[0] + p_ref[1]
    h = jnp.dot(h, w1_ref[...], preferred_element_type=jnp.float32)
    h = jnp.maximum(h + b1_ref[...], 0.0)
    h = jnp.dot(h, w2_ref[...], preferred_element_type=jnp.float32)
    h = jnp.maximum(h + b2_ref[...], 0.0)
    m = jnp.max(h, axis=-1, keepdims=True)
    lse = jnp.log(jnp.sum(jnp.exp(h - m), axis=-1, keepdims=True)) + m
    o_ref[...] = h - lse


BLK = 2000


def kernel(x, edge_index, eps, W1, b1, W2, b2):
    src = jnp.concatenate(
        [edge_index[0], jnp.zeros((E_PAD - E,), jnp.int32)])
    dst = jnp.concatenate(
        [edge_index[1], jnp.full((E_PAD - E,), N, jnp.int32)])
    zrows = jnp.zeros((ZROWS, D), jnp.float32)

    partials = _sc_call(src, dst, x, zrows)
    partials = partials.reshape(NC, N_PAD, D)

    grid = N // BLK
    out = pl.pallas_call(
        _tc_mlp,
        grid=(grid,),
        in_specs=[
            pl.BlockSpec(memory_space=pltpu.SMEM),
            pl.BlockSpec((BLK, D), lambda i: (i, 0)),
            pl.BlockSpec((NC, BLK, D), lambda i: (0, i, 0)),
            pl.BlockSpec((D, D), lambda i: (0, 0)),
            pl.BlockSpec((1, D), lambda i: (0, 0)),
            pl.BlockSpec((D, D), lambda i: (0, 0)),
            pl.BlockSpec((1, D), lambda i: (0, 0)),
        ],
        out_specs=pl.BlockSpec((BLK, D), lambda i: (i, 0)),
        out_shape=jax.ShapeDtypeStruct((N, D), jnp.float32),
    )(eps.reshape(1), x, partials, W1, b1.reshape(1, D), W2,
      b2.reshape(1, D))
    return out


# speedup vs baseline: 10.6471x; 2.3782x over previous
import functools

import jax
import jax.numpy as jnp
from jax import lax
from jax.experimental import pallas as pl
from jax.experimental.pallas import tpu as pltpu
from jax.experimental.pallas import tpu_sc as plsc

N = 10000
E = 320000
D = 128

NC = 2
NS = 16
NW = NC * NS

CHUNK = 128
EDGES_PER_TILE = E // NW
FULL_CHUNKS = EDGES_PER_TILE // CHUNK
TAIL = EDGES_PER_TILE - FULL_CHUNKS * CHUNK
GROUPS = (FULL_CHUNKS - 6) // 4
EPI0 = GROUPS * 4
ZTILES = 10
ZROWS = N // ZTILES


def _sc_aggregate(src_hbm, dst_hbm, x_hbm, z_hbm, out_hbm,
                  acc, sidx, didx, rows, tsidx, tdidx, trows,
                  gs0, gs1, ss0, ss1, is0, is1, is2, is3, tis, tgs, tss):
    gsems = (gs0, gs1)
    ssems = (ss0, ss1)
    isems = (is0, is1, is2, is3)
    cid = lax.axis_index("c")
    sid = lax.axis_index("s")
    wid = cid * NS + sid
    ebase = wid * EDGES_PER_TILE

    def idx_start(j, s):
        off = pl.multiple_of(ebase + j * CHUNK, 8)
        pltpu.async_copy(src_hbm.at[pl.ds(off, CHUNK)], sidx.at[s], isems[s])
        pltpu.async_copy(dst_hbm.at[pl.ds(off, CHUNK)], didx.at[s], isems[s])

    def idx_wait(s):
        pltpu.make_async_copy(src_hbm.at[pl.ds(0, CHUNK)], sidx.at[s],
                              isems[s]).wait()
        pltpu.make_async_copy(dst_hbm.at[pl.ds(0, CHUNK)], didx.at[s],
                              isems[s]).wait()

    def gather_start(b, s):
        pltpu.async_copy(x_hbm.at[sidx.at[s]], rows.at[b], gsems[b])

    def gather_wait(b):
        pltpu.make_async_copy(x_hbm.at[sidx.at[0]], rows.at[b],
                              gsems[b]).wait()

    def scatter_start(b, s):
        pltpu.async_copy(rows.at[b], acc.at[didx.at[s]], ssems[b], add=True)

    def scatter_wait(b):
        pltpu.make_async_copy(rows.at[b], acc.at[didx.at[0]],
                              ssems[b]).wait()

    for s in range(4):
        idx_start(s, s)

    @pl.when(sid < ZTILES)
    def _zero():
        pltpu.sync_copy(z_hbm, acc.at[pl.ds(sid * ZROWS, ZROWS)])

    idx_wait(0)
    gather_start(0, 0)
    idx_wait(1)
    gather_start(1, 1)
    plsc.subcore_barrier()

    def group(g, carry):
        t = g * 4
        gather_wait(0)
        scatter_start(0, 0)
        gather_wait(1)
        scatter_start(1, 1)
        scatter_wait(0)
        idx_start(t + 4, 0)
        idx_wait(2)
        gather_start(0, 2)
        scatter_wait(1)
        idx_start(t + 5, 1)
        idx_wait(3)
        gather_start(1, 3)
        gather_wait(0)
        scatter_start(0, 2)
        gather_wait(1)
        scatter_start(1, 3)
        scatter_wait(0)
        idx_start(t + 6, 2)
        idx_wait(0)
        gather_start(0, 0)
        scatter_wait(1)
        idx_start(t + 7, 3)
        idx_wait(1)
        gather_start(1, 1)
        return carry

    lax.fori_loop(0, GROUPS, group, 0)

    gather_wait(0)
    scatter_start(0, 0)
    gather_wait(1)
    scatter_start(1, 1)
    scatter_wait(0)
    idx_start(EPI0 + 4, 0)
    idx_wait(2)
    gather_start(0, 2)
    scatter_wait(1)
    idx_start(EPI0 + 5, 1)
    idx_wait(3)
    gather_start(1, 3)
    gather_wait(0)
    scatter_start(0, 2)
    gather_wait(1)
    scatter_start(1, 3)
    scatter_wait(0)
    idx_wait(0)
    gather_start(0, 0)
    scatter_wait(1)
    idx_wait(1)
    gather_start(1, 1)
    gather_wait(0)
    scatter_start(0, 0)
    gather_wait(1)
    scatter_start(1, 1)

    toff = pl.multiple_of(ebase + FULL_CHUNKS * CHUNK, 8)
    pltpu.async_copy(src_hbm.at[pl.ds(toff, TAIL)], tsidx, tis)
    pltpu.async_copy(dst_hbm.at[pl.ds(toff, TAIL)], tdidx, tis)
    pltpu.make_async_copy(src_hbm.at[pl.ds(0, TAIL)], tsidx, tis).wait()
    pltpu.make_async_copy(dst_hbm.at[pl.ds(0, TAIL)], tdidx, tis).wait()
    pltpu.async_copy(x_hbm.at[tsidx], trows, tgs).wait()
    pltpu.async_copy(trows, acc.at[tdidx], tss, add=True)
    scatter_wait(0)
    scatter_wait(1)
    pltpu.make_async_copy(trows, acc.at[tdidx], tss).wait()

    plsc.subcore_barrier()

    @pl.when(sid < ZTILES)
    def _writeout():
        row0 = sid * ZROWS
        out_off = pl.multiple_of(cid * N + row0, 8)
        pltpu.sync_copy(acc.at[pl.ds(row0, ZROWS)],
                        out_hbm.at[pl.ds(out_off, ZROWS)])


_sc_call = functools.partial(
    pl.kernel,
    out_type=jax.ShapeDtypeStruct((NC * N, D), jnp.float32),
    mesh=plsc.VectorSubcoreMesh(core_axis_name="c", subcore_axis_name="s",
                                num_cores=NC, num_subcores=NS),
    scratch_types=[
        pltpu.VMEM_SHARED((N, D), jnp.float32),
        pltpu.VMEM((4, CHUNK), jnp.int32),
        pltpu.VMEM((4, CHUNK), jnp.int32),
        pltpu.VMEM((2, CHUNK, D), jnp.float32),
        pltpu.VMEM((TAIL,), jnp.int32),
        pltpu.VMEM((TAIL,), jnp.int32),
        pltpu.VMEM((TAIL, D), jnp.float32),
    ] + [pltpu.SemaphoreType.DMA] * 11,
)(_sc_aggregate)


def _tc_mlp(eps_ref, x_ref, p_ref, w1_ref, b1_ref, w2_ref, b2_ref, o_ref):
    h = x_ref[...] * (1.0 + eps_ref[0]) + p_ref[0] + p_ref[1]
    h = jnp.dot(h, w1_ref[...], preferred_element_type=jnp.float32)
    h = jnp.maximum(h + b1_ref[...], 0.0)
    h = jnp.dot(h, w2_ref[...], preferred_element_type=jnp.float32)
    h = jnp.maximum(h + b2_ref[...], 0.0)
    m = jnp.max(h, axis=-1, keepdims=True)
    lse = jnp.log(jnp.sum(jnp.exp(h - m), axis=-1, keepdims=True)) + m
    o_ref[...] = h - lse


BLK = 2000


def kernel(x, edge_index, eps, W1, b1, W2, b2):
    src = edge_index[0]
    dst = edge_index[1]
    zrows = jnp.zeros((ZROWS, D), jnp.float32)

    partials = _sc_call(src, dst, x, zrows)
    partials = partials.reshape(NC, N, D)

    grid = N // BLK
    out = pl.pallas_call(
        _tc_mlp,
        grid=(grid,),
        in_specs=[
            pl.BlockSpec(memory_space=pltpu.SMEM),
            pl.BlockSpec((BLK, D), lambda i: (i, 0)),
            pl.BlockSpec((NC, BLK, D), lambda i: (0, i, 0)),
            pl.BlockSpec((D, D), lambda i: (0, 0)),
            pl.BlockSpec((1, D), lambda i: (0, 0)),
            pl.BlockSpec((D, D), lambda i: (0, 0)),
            pl.BlockSpec((1, D), lambda i: (0, 0)),
        ],
        out_specs=pl.BlockSpec((BLK, D), lambda i: (i, 0)),
        out_shape=jax.ShapeDtypeStruct((N, D), jnp.float32),
    )(eps.reshape(1), x, partials, W1, b1.reshape(1, D), W2,
      b2.reshape(1, D))
    return out
